# baseline (speedup 1.0000x reference)
import functools

import jax
import jax.numpy as jnp
from jax import lax
from jax.experimental import pallas as pl
from jax.experimental.pallas import tpu as pltpu
from jax.experimental.pallas import tpu_sc as plsc

B = 128
V = 100000
NC = 2
NS = 16
L = 16
NW = NC * NS
RPW = B // NW
CH = 10000
NCH = V // CH
TOT = RPW * NCH
VREGS = CH // L
UNROLL = 5
INNER = VREGS // UNROLL


def _sc_body(flat_hbm, tok_hbm, out_hbm,
             idx_v, vtmp_i, vtmp_f, buf0, buf1, red_v, res_v,
             sem0, sem1, gsem):
    wid = lax.axis_index("s") * NC + lax.axis_index("c")
    row0 = wid * RPW
    iota = lax.iota(jnp.int32, L)
    zero_i = jnp.zeros((L,), jnp.int32)

    vsplat = []
    for r in range(RPW):
        idx_v[...] = zero_i + (row0 + r)
        pltpu.async_copy(tok_hbm.at[idx_v], vtmp_i, gsem).wait()
        idx_v[...] = (row0 + r) * V + vtmp_i[...]
        pltpu.async_copy(flat_hbm.at[idx_v], vtmp_f, gsem).wait()
        vsplat.append(vtmp_f[...])

    def dma(g, buf, sem):
        r = g // NCH
        c = g - r * NCH
        base = (row0 + r) * V + c * CH
        return pltpu.make_async_copy(flat_hbm.at[pl.ds(base, CH)], buf, sem)

    dma(0, buf0, sem0).start()
    dma(1, buf1, sem1).start()
    red_v[pl.ds(0, L)] = zero_i
    red_v[pl.ds(L, L)] = zero_i
    for k in range(8):
        res_v[pl.ds(k * L, L)] = zero_i

    def count_one(g, buf, sem, acc):
        dma(g, buf, sem).wait()

        @pl.when(g + 2 < TOT)
        def _():
            dma(g + 2, buf, sem).start()

        r = g // NCH
        val = jnp.where(
            r == 0, vsplat[0],
            jnp.where(r == 1, vsplat[1],
                      jnp.where(r == 2, vsplat[2], vsplat[3])))

        def inner(i, a):
            for u in range(UNROLL):
                x = buf[pl.ds((i * UNROLL + u) * L, L)]
                a = a + jnp.where(x > val, 1, 0).astype(jnp.int32)
            return a

        acc = lax.fori_loop(0, INNER, inner, acc)
        row_end = g - r * NCH == NCH - 1

        @pl.when(row_end)
        def _():
            red_v[pl.ds(0, L)] = acc
            a = acc + red_v[pl.ds(8, L)]
            red_v[pl.ds(0, L)] = a
            a = a + red_v[pl.ds(4, L)]
            red_v[pl.ds(0, L)] = a
            a = a + red_v[pl.ds(2, L)]
            red_v[pl.ds(0, L)] = a
            a = a + red_v[pl.ds(1, L)]
            res_v[pl.ds(r, L)] = a + 1

        return jnp.where(row_end, 0, acc)

    def body2(h, acc):
        acc = count_one(2 * h, buf0, sem0, acc)
        acc = count_one(2 * h + 1, buf1, sem1, acc)
        return acc

    lax.fori_loop(0, TOT // 2, body2, zero_i)
    pltpu.sync_copy(res_v, out_hbm.at[wid])


_ranks_sc = functools.partial(
    pl.kernel,
    out_type=jax.ShapeDtypeStruct((NW, 128), jnp.int32),
    mesh=plsc.VectorSubcoreMesh(core_axis_name="c", subcore_axis_name="s"),
    scratch_types=[
        pltpu.VMEM((L,), jnp.int32),
        pltpu.VMEM((L,), jnp.int32),
        pltpu.VMEM((L,), jnp.float32),
        pltpu.VMEM((CH,), jnp.float32),
        pltpu.VMEM((CH,), jnp.float32),
        pltpu.VMEM((2 * L,), jnp.int32),
        pltpu.VMEM((128,), jnp.int32),
        pltpu.SemaphoreType.DMA,
        pltpu.SemaphoreType.DMA,
        pltpu.SemaphoreType.DMA,
    ],
)(_sc_body)


def kernel(logits, token_ids):
    flat = logits.reshape(B * V)
    tok = token_ids.astype(jnp.int32)
    out = _ranks_sc(flat, tok)
    return out[:, :RPW].reshape(B).astype(jnp.int64)
